# baseline (speedup 1.0000x reference)
import functools

import jax
import jax.numpy as jnp
from jax import lax
from jax.experimental import pallas as pl
from jax.experimental.pallas import tpu as pltpu
from jax.experimental.pallas import tpu_sc as plsc

B = 8
L2 = 3
N2 = 512
L1 = 6
N1 = 2048
IN = 256
MID = 128
OUT = 128
ORIG = 64
BN_EPS = 1e-5

SC_NC = 2
SC_NS = 16
SC_NW = SC_NC * SC_NS
N_ITEMS = B * L1 * 2
APW = N1 // 2
CHUNK = 128
N_CHUNK = APW // CHUNK
N_GRP = CHUNK // 16


def _stage1_body(feat_ref, wt_ref, wmid_ref, gamma_ref, beta_ref, h_ref):
    wt = wt_ref[0]
    s1 = jnp.zeros((MID, 1), jnp.float32)
    s2 = jnp.zeros((MID, 1), jnp.float32)
    for b in range(B):
        g = jnp.dot(wt, feat_ref[b, 0], preferred_element_type=jnp.float32)
        h_ref[0, b] = g
        s1 = s1 + jnp.sum(g, axis=1, keepdims=True)
        s2 = s2 + jnp.sum(g * g, axis=1, keepdims=True)
    inv_n = jnp.float32(1.0 / (B * N2))
    mean = s1 * inv_n
    var = s2 * inv_n - mean * mean
    rstd = lax.rsqrt(var + BN_EPS)
    scale = gamma_ref[...] * rstd
    bias = beta_ref[...] - mean * scale
    wmid = wmid_ref[...]
    for b in range(B):
        sf = jnp.maximum(h_ref[0, b] * scale + bias, 0.0)
        h_ref[0, b] = jnp.dot(wmid, sf, preferred_element_type=jnp.float32)


def _stage2_body(seed_ref, anchor_ref, orig_ref, worig_ref,
                 idx_ref, w_ref, part_ref):
    s = seed_ref[0, 0]
    a = anchor_ref[0, 0]
    dx = s[:, 0:1] - a[0:1, :]
    dy = s[:, 1:2] - a[1:2, :]
    dz = s[:, 2:3] - a[2:3, :]
    d2 = (dx * dx + dy * dy) + dz * dz

    miota = lax.broadcasted_iota(jnp.int32, (1, N2), 1).astype(jnp.float32)
    recips = []
    for k in range(3):
        mn = jnp.min(d2, axis=0, keepdims=True)
        sel = d2 == mn
        recips.append(1.0 / (mn + 1e-8))
        idx_f = jnp.dot(miota, sel.astype(jnp.float32),
                        preferred_element_type=jnp.float32,
                        precision=lax.Precision.HIGHEST)
        idx_ref[0, 0, k:k + 1, :] = (idx_f + 0.5).astype(jnp.int32)
        d2 = jnp.where(sel, jnp.float32(jnp.inf), d2)

    norm = (recips[0] + recips[1]) + recips[2]
    for k in range(3):
        w_ref[0, 0, k:k + 1, :] = recips[k] / norm

    part_ref[0, 0] = jnp.dot(worig_ref[...], orig_ref[0, 0],
                             preferred_element_type=jnp.float32)


def _sc_gather_body(ht_hbm, idx_hbm, w_hbm, part_hbm, out_hbm,
                    table_v, idx_v, w_v, tile_v):
    wid = lax.axis_index("s") * SC_NC + lax.axis_index("c")
    for it in range(N_ITEMS // SC_NW):
        item = wid * (N_ITEMS // SC_NW) + it
        s = item // 2
        half = item % 2
        b = s // L1
        j = s % L1
        abase = half * APW
        pltpu.sync_copy(ht_hbm.at[b, j], table_v)
        pltpu.sync_copy(idx_hbm.at[b, j, :, pl.ds(abase, APW)], idx_v)
        pltpu.sync_copy(w_hbm.at[b, j, :, pl.ds(abase, APW)], w_v)
        for cc in range(N_CHUNK):
            a0 = abase + cc * CHUNK
            pltpu.sync_copy(part_hbm.at[b, j, :, pl.ds(a0, CHUNK)], tile_v)
            for g in range(N_GRP):
                o = cc * CHUNK + g * 16
                a0g = idx_v[0, pl.ds(o, 16)] * MID
                a1g = idx_v[1, pl.ds(o, 16)] * MID
                a2g = idx_v[2, pl.ds(o, 16)] * MID
                w0 = w_v[0, pl.ds(o, 16)]
                w1 = w_v[1, pl.ds(o, 16)]
                w2 = w_v[2, pl.ds(o, 16)]

                def chan_body(c, _):
                    v = w0 * plsc.load_gather(table_v, [a0g + c])
                    v = v + w1 * plsc.load_gather(table_v, [a1g + c])
                    v = v + w2 * plsc.load_gather(table_v, [a2g + c])
                    plsc.addupdate(tile_v.at[c, pl.ds(g * 16, 16)], v)
                    return 0

                lax.fori_loop(0, MID, chan_body, 0)
            pltpu.sync_copy(tile_v, out_hbm.at[b, j, :, pl.ds(a0, CHUNK)])


@jax.jit
def kernel(xyzs, original_xyzs, features, original_features, W_temporal,
           bn_gamma, bn_beta, W_spatial):
    w_taps = W_temporal.reshape(3, MID, IN)
    w_mid = W_spatial[:, :MID]
    w_orig = W_spatial[:, MID:]
    gamma = bn_gamma.reshape(MID, 1)
    beta = bn_beta.reshape(MID, 1)
    anchors_t = jnp.swapaxes(original_xyzs, 2, 3)

    h = pl.pallas_call(
        _stage1_body,
        grid=(L1,),
        in_specs=[
            pl.BlockSpec((B, 1, IN, N2), lambda j: (0, j // 2, 0, 0)),
            pl.BlockSpec((1, MID, IN), lambda j: (j % 2, 0, 0)),
            pl.BlockSpec((MID, MID), lambda j: (0, 0)),
            pl.BlockSpec((MID, 1), lambda j: (0, 0)),
            pl.BlockSpec((MID, 1), lambda j: (0, 0)),
        ],
        out_specs=pl.BlockSpec((1, B, MID, N2), lambda j: (j, 0, 0, 0)),
        out_shape=jax.ShapeDtypeStruct((L1, B, MID, N2), jnp.float32),
    )(features, w_taps, w_mid, gamma, beta)

    h_t = jnp.transpose(h, (1, 0, 3, 2)).reshape(B, L1, N2 * MID)

    idx3, w3, partial = pl.pallas_call(
        _stage2_body,
        grid=(B, L1),
        in_specs=[
            pl.BlockSpec((1, 1, N2, 3), lambda b, j: (b, j // 2, 0, 0)),
            pl.BlockSpec((1, 1, 3, N1), lambda b, j: (b, j, 0, 0)),
            pl.BlockSpec((1, 1, ORIG, N1), lambda b, j: (b, j, 0, 0)),
            pl.BlockSpec((OUT, ORIG), lambda b, j: (0, 0)),
        ],
        out_specs=[
            pl.BlockSpec((1, 1, 3, N1), lambda b, j: (b, j, 0, 0)),
            pl.BlockSpec((1, 1, 3, N1), lambda b, j: (b, j, 0, 0)),
            pl.BlockSpec((1, 1, OUT, N1), lambda b, j: (b, j, 0, 0)),
        ],
        out_shape=[
            jax.ShapeDtypeStruct((B, L1, 3, N1), jnp.int32),
            jax.ShapeDtypeStruct((B, L1, 3, N1), jnp.float32),
            jax.ShapeDtypeStruct((B, L1, OUT, N1), jnp.float32),
        ],
    )(xyzs, anchors_t, original_features, w_orig)

    sc_call = functools.partial(
        pl.kernel,
        mesh=plsc.VectorSubcoreMesh(core_axis_name="c", subcore_axis_name="s"),
        out_type=jax.ShapeDtypeStruct((B, L1, OUT, N1), jnp.float32),
        scratch_types=[
            pltpu.VMEM((N2 * MID,), jnp.float32),
            pltpu.VMEM((3, APW), jnp.int32),
            pltpu.VMEM((3, APW), jnp.float32),
            pltpu.VMEM((OUT, CHUNK), jnp.float32),
        ],
        compiler_params=pltpu.CompilerParams(needs_layout_passes=False),
    )(_sc_gather_body)

    new_features = sc_call(h_t, idx3, w3, partial)
    return original_xyzs, new_features


# speedup vs baseline: 1.2515x; 1.2515x over previous
import functools

import jax
import jax.numpy as jnp
from jax import lax
from jax.experimental import pallas as pl
from jax.experimental.pallas import tpu as pltpu
from jax.experimental.pallas import tpu_sc as plsc

B = 8
L2 = 3
N2 = 512
L1 = 6
N1 = 2048
IN = 256
MID = 128
OUT = 128
ORIG = 64
BN_EPS = 1e-5

SC_NC = 2
SC_NS = 16
SC_NW = SC_NC * SC_NS
N_ITEMS = B * L1 * 2
APW = N1 // 2
CHUNK = 128
N_CHUNK = APW // CHUNK
N_GRP = CHUNK // 16


def _stage1_body(feat_ref, wt_ref, wmid_ref, gamma_ref, beta_ref, h_ref):
    wt = wt_ref[0]
    s1 = jnp.zeros((MID, 1), jnp.float32)
    s2 = jnp.zeros((MID, 1), jnp.float32)
    for b in range(B):
        g = jnp.dot(wt, feat_ref[b, 0], preferred_element_type=jnp.float32)
        h_ref[0, b] = g
        s1 = s1 + jnp.sum(g, axis=1, keepdims=True)
        s2 = s2 + jnp.sum(g * g, axis=1, keepdims=True)
    inv_n = jnp.float32(1.0 / (B * N2))
    mean = s1 * inv_n
    var = s2 * inv_n - mean * mean
    rstd = lax.rsqrt(var + BN_EPS)
    scale = gamma_ref[...] * rstd
    bias = beta_ref[...] - mean * scale
    wmid = wmid_ref[...]
    for b in range(B):
        sf = jnp.maximum(h_ref[0, b] * scale + bias, 0.0)
        h_ref[0, b] = jnp.dot(wmid, sf, preferred_element_type=jnp.float32)


def _stage2_body(seed_ref, anchor_ref, orig_ref, worig_ref,
                 idx_ref, w_ref, part_ref):
    s = seed_ref[0, 0]
    a = anchor_ref[0, 0]
    dx = s[:, 0:1] - a[0:1, :]
    dy = s[:, 1:2] - a[1:2, :]
    dz = s[:, 2:3] - a[2:3, :]
    d2 = (dx * dx + dy * dy) + dz * dz

    miota = lax.broadcasted_iota(jnp.int32, (1, N2), 1).astype(jnp.float32)
    recips = []
    for k in range(3):
        mn = jnp.min(d2, axis=0, keepdims=True)
        sel = d2 == mn
        recips.append(1.0 / (mn + 1e-8))
        idx_f = jnp.dot(miota, sel.astype(jnp.float32),
                        preferred_element_type=jnp.float32,
                        precision=lax.Precision.HIGHEST)
        idx_ref[0, 0, k:k + 1, :] = (idx_f + 0.5).astype(jnp.int32)
        d2 = jnp.where(sel, jnp.float32(jnp.inf), d2)

    norm = (recips[0] + recips[1]) + recips[2]
    for k in range(3):
        w_ref[0, 0, k:k + 1, :] = recips[k] / norm

    part_ref[0, 0] = jnp.dot(worig_ref[...], orig_ref[0, 0],
                             preferred_element_type=jnp.float32)


def _sc_gather_body(ht_hbm, idx_hbm, w_hbm, part_hbm, out_hbm,
                    table_v, idx_v, w_v, tile_v):
    wid = lax.axis_index("s") * SC_NC + lax.axis_index("c")

    def item_body(it, _):
        item = wid * (N_ITEMS // SC_NW) + it
        s = item // 2
        half = item % 2
        b = s // L1
        j = s % L1
        abase = half * APW
        pltpu.sync_copy(ht_hbm.at[b, j], table_v)
        pltpu.sync_copy(idx_hbm.at[b, j, :, pl.ds(abase, APW)], idx_v)
        pltpu.sync_copy(w_hbm.at[b, j, :, pl.ds(abase, APW)], w_v)

        def chunk_body(cc, _2):
            a0 = abase + cc * CHUNK
            pltpu.sync_copy(part_hbm.at[b, j, :, pl.ds(a0, CHUNK)], tile_v)
            for g in range(N_GRP):
                o = cc * CHUNK + g * 16
                a0g = idx_v[0, pl.ds(o, 16)] * MID
                a1g = idx_v[1, pl.ds(o, 16)] * MID
                a2g = idx_v[2, pl.ds(o, 16)] * MID
                w0 = w_v[0, pl.ds(o, 16)]
                w1 = w_v[1, pl.ds(o, 16)]
                w2 = w_v[2, pl.ds(o, 16)]

                @plsc.parallel_loop(0, MID, 1, unroll=8)
                def chan_body(c):
                    v = w0 * plsc.load_gather(table_v, [a0g + c])
                    v = v + w1 * plsc.load_gather(table_v, [a1g + c])
                    v = v + w2 * plsc.load_gather(table_v, [a2g + c])
                    plsc.addupdate(tile_v.at[c, pl.ds(g * 16, 16)], v)
            pltpu.sync_copy(tile_v, out_hbm.at[b, j, :, pl.ds(a0, CHUNK)])
            return 0

        lax.fori_loop(0, N_CHUNK, chunk_body, 0)
        return 0

    lax.fori_loop(0, N_ITEMS // SC_NW, item_body, 0)


@jax.jit
def kernel(xyzs, original_xyzs, features, original_features, W_temporal,
           bn_gamma, bn_beta, W_spatial):
    w_taps = W_temporal.reshape(3, MID, IN)
    w_mid = W_spatial[:, :MID]
    w_orig = W_spatial[:, MID:]
    gamma = bn_gamma.reshape(MID, 1)
    beta = bn_beta.reshape(MID, 1)
    anchors_t = jnp.swapaxes(original_xyzs, 2, 3)

    h = pl.pallas_call(
        _stage1_body,
        grid=(L1,),
        in_specs=[
            pl.BlockSpec((B, 1, IN, N2), lambda j: (0, j // 2, 0, 0)),
            pl.BlockSpec((1, MID, IN), lambda j: (j % 2, 0, 0)),
            pl.BlockSpec((MID, MID), lambda j: (0, 0)),
            pl.BlockSpec((MID, 1), lambda j: (0, 0)),
            pl.BlockSpec((MID, 1), lambda j: (0, 0)),
        ],
        out_specs=pl.BlockSpec((1, B, MID, N2), lambda j: (j, 0, 0, 0)),
        out_shape=jax.ShapeDtypeStruct((L1, B, MID, N2), jnp.float32),
    )(features, w_taps, w_mid, gamma, beta)

    h_t = jnp.transpose(h, (1, 0, 3, 2)).reshape(B, L1, N2 * MID)

    idx3, w3, partial = pl.pallas_call(
        _stage2_body,
        grid=(B, L1),
        in_specs=[
            pl.BlockSpec((1, 1, N2, 3), lambda b, j: (b, j // 2, 0, 0)),
            pl.BlockSpec((1, 1, 3, N1), lambda b, j: (b, j, 0, 0)),
            pl.BlockSpec((1, 1, ORIG, N1), lambda b, j: (b, j, 0, 0)),
            pl.BlockSpec((OUT, ORIG), lambda b, j: (0, 0)),
        ],
        out_specs=[
            pl.BlockSpec((1, 1, 3, N1), lambda b, j: (b, j, 0, 0)),
            pl.BlockSpec((1, 1, 3, N1), lambda b, j: (b, j, 0, 0)),
            pl.BlockSpec((1, 1, OUT, N1), lambda b, j: (b, j, 0, 0)),
        ],
        out_shape=[
            jax.ShapeDtypeStruct((B, L1, 3, N1), jnp.int32),
            jax.ShapeDtypeStruct((B, L1, 3, N1), jnp.float32),
            jax.ShapeDtypeStruct((B, L1, OUT, N1), jnp.float32),
        ],
    )(xyzs, anchors_t, original_features, w_orig)

    sc_call = functools.partial(
        pl.kernel,
        mesh=plsc.VectorSubcoreMesh(core_axis_name="c", subcore_axis_name="s"),
        out_type=jax.ShapeDtypeStruct((B, L1, OUT, N1), jnp.float32),
        scratch_types=[
            pltpu.VMEM((N2 * MID,), jnp.float32),
            pltpu.VMEM((3, APW), jnp.int32),
            pltpu.VMEM((3, APW), jnp.float32),
            pltpu.VMEM((OUT, CHUNK), jnp.float32),
        ],
        compiler_params=pltpu.CompilerParams(needs_layout_passes=False),
    )(_sc_gather_body)

    new_features = sc_call(h_t, idx3, w3, partial)
    return original_xyzs, new_features


# speedup vs baseline: 2.2086x; 1.7647x over previous
import functools

import jax
import jax.numpy as jnp
from jax import lax
from jax.experimental import pallas as pl
from jax.experimental.pallas import tpu as pltpu
from jax.experimental.pallas import tpu_sc as plsc

B = 8
L2 = 3
N2 = 512
L1 = 6
N1 = 2048
IN = 256
MID = 128
OUT = 128
ORIG = 64
BN_EPS = 1e-5

SC_NC = 2
SC_NS = 16
SC_NW = SC_NC * SC_NS
CPI = 16
N_SLOT = MID // CPI
N_ITEMS = B * L1 * N_SLOT
ITEMS_PW = N_ITEMS // SC_NW
N_GRP = N1 // 16


def _stage1_body(feat_ref, wt_ref, wmid_ref, gamma_ref, beta_ref, h_ref):
    wt = wt_ref[0]
    s1 = jnp.zeros((MID, 1), jnp.float32)
    s2 = jnp.zeros((MID, 1), jnp.float32)
    for b in range(B):
        g = jnp.dot(wt, feat_ref[b, 0], preferred_element_type=jnp.float32)
        h_ref[0, b] = g
        s1 = s1 + jnp.sum(g, axis=1, keepdims=True)
        s2 = s2 + jnp.sum(g * g, axis=1, keepdims=True)
    inv_n = jnp.float32(1.0 / (B * N2))
    mean = s1 * inv_n
    var = s2 * inv_n - mean * mean
    rstd = lax.rsqrt(var + BN_EPS)
    scale = gamma_ref[...] * rstd
    bias = beta_ref[...] - mean * scale
    wmid = wmid_ref[...]
    for b in range(B):
        sf = jnp.maximum(h_ref[0, b] * scale + bias, 0.0)
        h_ref[0, b] = jnp.dot(wmid, sf, preferred_element_type=jnp.float32)


def _stage2_body(seed_ref, anchor_ref, orig_ref, worig_ref,
                 idx_ref, w_ref, part_ref):
    s = seed_ref[0, 0]
    a = anchor_ref[0, 0]
    dx = s[:, 0:1] - a[0:1, :]
    dy = s[:, 1:2] - a[1:2, :]
    dz = s[:, 2:3] - a[2:3, :]
    d2 = (dx * dx + dy * dy) + dz * dz

    iota_i = lax.broadcasted_iota(jnp.int32, (1, N2), 1)
    mlow = (iota_i & 15).astype(jnp.float32)
    mhigh = (iota_i >> 4).astype(jnp.float32)
    recips = []
    for k in range(3):
        mn = jnp.min(d2, axis=0, keepdims=True)
        sel = d2 == mn
        recips.append(1.0 / (mn + 1e-8))
        sel_f = sel.astype(jnp.float32)
        il = jnp.dot(mlow, sel_f, preferred_element_type=jnp.float32)
        ih = jnp.dot(mhigh, sel_f, preferred_element_type=jnp.float32)
        idx_f = il + 16.0 * ih
        idx_i = (idx_f + 0.5).astype(jnp.int32)
        idx_ref[0, 0, k:k + 1, :] = jnp.minimum(idx_i, N2 - 1)
        d2 = jnp.where(sel, jnp.float32(jnp.inf), d2)

    norm = (recips[0] + recips[1]) + recips[2]
    for k in range(3):
        w_ref[0, 0, k:k + 1, :] = recips[k] / norm

    part_ref[0, 0] = jnp.dot(worig_ref[...], orig_ref[0, 0],
                             preferred_element_type=jnp.float32)


def _sc_gather_body(ht_hbm, idx_hbm, w_hbm, part_hbm, out_hbm,
                    table_v, idx_v, w_v, tile_v):
    wid = lax.axis_index("s") * SC_NC + lax.axis_index("c")

    def item_body(it, _):
        item = wid * ITEMS_PW + it
        s = item // N_SLOT
        cslot = item % N_SLOT
        b = s // L1
        j = s % L1
        pltpu.sync_copy(ht_hbm.at[j, b, cslot], table_v)
        pltpu.sync_copy(idx_hbm.at[b, j], idx_v)
        pltpu.sync_copy(w_hbm.at[b, j], w_v)
        pltpu.sync_copy(part_hbm.at[b, j, cslot], tile_v)

        @plsc.parallel_loop(0, N_GRP, 1, unroll=2)
        def grp_body(g):
            o = g * 16
            i0 = idx_v[0, pl.ds(o, 16)]
            i1 = idx_v[1, pl.ds(o, 16)]
            i2 = idx_v[2, pl.ds(o, 16)]
            w0 = w_v[0, pl.ds(o, 16)]
            w1 = w_v[1, pl.ds(o, 16)]
            w2 = w_v[2, pl.ds(o, 16)]
            for c in range(CPI):
                v = w0 * plsc.load_gather(table_v, [i0 + c * N2])
                v = v + w1 * plsc.load_gather(table_v, [i1 + c * N2])
                v = v + w2 * plsc.load_gather(table_v, [i2 + c * N2])
                plsc.addupdate(tile_v.at[pl.ds(c * N1 + o, 16)], v)

        pltpu.sync_copy(tile_v, out_hbm.at[b, j, cslot])
        return 0

    lax.fori_loop(0, ITEMS_PW, item_body, 0)


@jax.jit
def kernel(xyzs, original_xyzs, features, original_features, W_temporal,
           bn_gamma, bn_beta, W_spatial):
    w_taps = W_temporal.reshape(3, MID, IN)
    w_mid = W_spatial[:, :MID]
    w_orig = W_spatial[:, MID:]
    gamma = bn_gamma.reshape(MID, 1)
    beta = bn_beta.reshape(MID, 1)
    anchors_t = jnp.swapaxes(original_xyzs, 2, 3)

    h = pl.pallas_call(
        _stage1_body,
        grid=(L1,),
        in_specs=[
            pl.BlockSpec((B, 1, IN, N2), lambda j: (0, j // 2, 0, 0)),
            pl.BlockSpec((1, MID, IN), lambda j: (j % 2, 0, 0)),
            pl.BlockSpec((MID, MID), lambda j: (0, 0)),
            pl.BlockSpec((MID, 1), lambda j: (0, 0)),
            pl.BlockSpec((MID, 1), lambda j: (0, 0)),
        ],
        out_specs=pl.BlockSpec((1, B, MID, N2), lambda j: (j, 0, 0, 0)),
        out_shape=jax.ShapeDtypeStruct((L1, B, MID, N2), jnp.float32),
    )(features, w_taps, w_mid, gamma, beta)

    h_t = h.reshape(L1, B, N_SLOT, CPI * N2)

    idx3, w3, partial = pl.pallas_call(
        _stage2_body,
        grid=(B, L1),
        in_specs=[
            pl.BlockSpec((1, 1, N2, 3), lambda b, j: (b, j // 2, 0, 0)),
            pl.BlockSpec((1, 1, 3, N1), lambda b, j: (b, j, 0, 0)),
            pl.BlockSpec((1, 1, ORIG, N1), lambda b, j: (b, j, 0, 0)),
            pl.BlockSpec((OUT, ORIG), lambda b, j: (0, 0)),
        ],
        out_specs=[
            pl.BlockSpec((1, 1, 3, N1), lambda b, j: (b, j, 0, 0)),
            pl.BlockSpec((1, 1, 3, N1), lambda b, j: (b, j, 0, 0)),
            pl.BlockSpec((1, 1, OUT, N1), lambda b, j: (b, j, 0, 0)),
        ],
        out_shape=[
            jax.ShapeDtypeStruct((B, L1, 3, N1), jnp.int32),
            jax.ShapeDtypeStruct((B, L1, 3, N1), jnp.float32),
            jax.ShapeDtypeStruct((B, L1, OUT, N1), jnp.float32),
        ],
    )(xyzs, anchors_t, original_features, w_orig)

    sc_call = functools.partial(
        pl.kernel,
        mesh=plsc.VectorSubcoreMesh(core_axis_name="c", subcore_axis_name="s"),
        out_type=jax.ShapeDtypeStruct((B, L1, N_SLOT, CPI * N1), jnp.float32),
        scratch_types=[
            pltpu.VMEM((CPI * N2,), jnp.float32),
            pltpu.VMEM((3, N1), jnp.int32),
            pltpu.VMEM((3, N1), jnp.float32),
            pltpu.VMEM((CPI * N1,), jnp.float32),
        ],
        compiler_params=pltpu.CompilerParams(needs_layout_passes=False),
    )(_sc_gather_body)

    part_r = partial.reshape(B, L1, N_SLOT, CPI * N1)
    new_features = sc_call(h_t, idx3, w3, part_r).reshape(B, L1, OUT, N1)
    return original_xyzs, new_features


# speedup vs baseline: 2.9227x; 1.3233x over previous
import functools

import jax
import jax.numpy as jnp
from jax import lax
from jax.experimental import pallas as pl
from jax.experimental.pallas import tpu as pltpu
from jax.experimental.pallas import tpu_sc as plsc

B = 8
L2 = 3
N2 = 512
L1 = 6
N1 = 2048
IN = 256
MID = 128
OUT = 128
ORIG = 64
BN_EPS = 1e-5

SC_NC = 2
SC_NS = 16
SC_NW = SC_NC * SC_NS
CPI = 16
N_SLOT = MID // CPI
N_ITEMS = B * L1 * N_SLOT
ITEMS_PW = N_ITEMS // SC_NW
N_GRP = N1 // 16


def _stage1_body(feat_ref, wt_ref, wmid_ref, gamma_ref, beta_ref, h_ref):
    wt = wt_ref[0]
    s1 = jnp.zeros((MID, 1), jnp.float32)
    s2 = jnp.zeros((MID, 1), jnp.float32)
    for b in range(B):
        g = jnp.dot(wt, feat_ref[b, 0], preferred_element_type=jnp.float32)
        h_ref[0, b] = g
        s1 = s1 + jnp.sum(g, axis=1, keepdims=True)
        s2 = s2 + jnp.sum(g * g, axis=1, keepdims=True)
    inv_n = jnp.float32(1.0 / (B * N2))
    mean = s1 * inv_n
    var = s2 * inv_n - mean * mean
    rstd = lax.rsqrt(var + BN_EPS)
    scale = gamma_ref[...] * rstd
    bias = beta_ref[...] - mean * scale
    wmid = wmid_ref[...]
    for b in range(B):
        sf = jnp.maximum(h_ref[0, b] * scale + bias, 0.0)
        h_ref[0, b] = jnp.dot(wmid, sf, preferred_element_type=jnp.float32)


def _stage2_body(seed_ref, anchor_ref, orig_ref, worig_ref,
                 idx_ref, w_ref, part_ref):
    s = seed_ref[0, 0]
    a = anchor_ref[0, 0]
    dx = s[:, 0:1] - a[0:1, :]
    dy = s[:, 1:2] - a[1:2, :]
    dz = s[:, 2:3] - a[2:3, :]
    d2 = (dx * dx + dy * dy) + dz * dz

    iota_i = lax.broadcasted_iota(jnp.int32, (1, N2), 1)
    mlow = (iota_i & 15).astype(jnp.float32)
    mhigh = (iota_i >> 4).astype(jnp.float32)
    recips = []
    for k in range(3):
        mn = jnp.min(d2, axis=0, keepdims=True)
        sel = d2 == mn
        recips.append(1.0 / (mn + 1e-8))
        sel_f = sel.astype(jnp.float32)
        il = jnp.dot(mlow, sel_f, preferred_element_type=jnp.float32)
        ih = jnp.dot(mhigh, sel_f, preferred_element_type=jnp.float32)
        idx_f = il + 16.0 * ih
        idx_i = (idx_f + 0.5).astype(jnp.int32)
        idx_ref[0, 0, k:k + 1, :] = jnp.minimum(idx_i, N2 - 1)
        d2 = jnp.where(sel, jnp.float32(jnp.inf), d2)

    norm = (recips[0] + recips[1]) + recips[2]
    for k in range(3):
        w_ref[0, 0, k:k + 1, :] = recips[k] / norm

    part_ref[0, 0] = jnp.dot(worig_ref[...], orig_ref[0, 0],
                             preferred_element_type=jnp.float32)


def _sc_gather_body(ht_hbm, idx_hbm, w_hbm, part_hbm, out_hbm,
                    table_v, idx_v, w_v, tile_v):
    wid = lax.axis_index("s") * SC_NC + lax.axis_index("c")

    def item_body(it, _):
        item = wid * ITEMS_PW + it
        s = item // N_SLOT
        cslot = item % N_SLOT
        b = s // L1
        j = s % L1
        c0 = cslot * CPI
        pltpu.sync_copy(ht_hbm.at[j, b, pl.ds(c0, CPI), :], table_v)
        pltpu.sync_copy(idx_hbm.at[b, j], idx_v)
        pltpu.sync_copy(w_hbm.at[b, j], w_v)
        pltpu.sync_copy(part_hbm.at[b, j, pl.ds(c0, CPI), :], tile_v)

        @plsc.parallel_loop(0, N_GRP, 1, unroll=2)
        def grp_body(g):
            o = g * 16
            i0 = idx_v[0, pl.ds(o, 16)]
            i1 = idx_v[1, pl.ds(o, 16)]
            i2 = idx_v[2, pl.ds(o, 16)]
            w0 = w_v[0, pl.ds(o, 16)]
            w1 = w_v[1, pl.ds(o, 16)]
            w2 = w_v[2, pl.ds(o, 16)]
            for c in range(CPI):
                cc = jnp.full((16,), c, jnp.int32)
                v = w0 * plsc.load_gather(table_v, [cc, i0])
                v = v + w1 * plsc.load_gather(table_v, [cc, i1])
                v = v + w2 * plsc.load_gather(table_v, [cc, i2])
                plsc.addupdate(tile_v.at[c, pl.ds(o, 16)], v)

        pltpu.sync_copy(tile_v, out_hbm.at[b, j, pl.ds(c0, CPI), :])
        return 0

    lax.fori_loop(0, ITEMS_PW, item_body, 0)


@jax.jit
def kernel(xyzs, original_xyzs, features, original_features, W_temporal,
           bn_gamma, bn_beta, W_spatial):
    w_taps = W_temporal.reshape(3, MID, IN)
    w_mid = W_spatial[:, :MID]
    w_orig = W_spatial[:, MID:]
    gamma = bn_gamma.reshape(MID, 1)
    beta = bn_beta.reshape(MID, 1)
    anchors_t = jnp.swapaxes(original_xyzs, 2, 3)

    h = pl.pallas_call(
        _stage1_body,
        grid=(L1,),
        in_specs=[
            pl.BlockSpec((B, 1, IN, N2), lambda j: (0, j // 2, 0, 0)),
            pl.BlockSpec((1, MID, IN), lambda j: (j % 2, 0, 0)),
            pl.BlockSpec((MID, MID), lambda j: (0, 0)),
            pl.BlockSpec((MID, 1), lambda j: (0, 0)),
            pl.BlockSpec((MID, 1), lambda j: (0, 0)),
        ],
        out_specs=pl.BlockSpec((1, B, MID, N2), lambda j: (j, 0, 0, 0)),
        out_shape=jax.ShapeDtypeStruct((L1, B, MID, N2), jnp.float32),
    )(features, w_taps, w_mid, gamma, beta)


    idx3, w3, partial = pl.pallas_call(
        _stage2_body,
        grid=(B, L1),
        in_specs=[
            pl.BlockSpec((1, 1, N2, 3), lambda b, j: (b, j // 2, 0, 0)),
            pl.BlockSpec((1, 1, 3, N1), lambda b, j: (b, j, 0, 0)),
            pl.BlockSpec((1, 1, ORIG, N1), lambda b, j: (b, j, 0, 0)),
            pl.BlockSpec((OUT, ORIG), lambda b, j: (0, 0)),
        ],
        out_specs=[
            pl.BlockSpec((1, 1, 3, N1), lambda b, j: (b, j, 0, 0)),
            pl.BlockSpec((1, 1, 3, N1), lambda b, j: (b, j, 0, 0)),
            pl.BlockSpec((1, 1, OUT, N1), lambda b, j: (b, j, 0, 0)),
        ],
        out_shape=[
            jax.ShapeDtypeStruct((B, L1, 3, N1), jnp.int32),
            jax.ShapeDtypeStruct((B, L1, 3, N1), jnp.float32),
            jax.ShapeDtypeStruct((B, L1, OUT, N1), jnp.float32),
        ],
    )(xyzs, anchors_t, original_features, w_orig)

    sc_call = functools.partial(
        pl.kernel,
        mesh=plsc.VectorSubcoreMesh(core_axis_name="c", subcore_axis_name="s"),
        out_type=jax.ShapeDtypeStruct((B, L1, OUT, N1), jnp.float32),
        scratch_types=[
            pltpu.VMEM((CPI, N2), jnp.float32),
            pltpu.VMEM((3, N1), jnp.int32),
            pltpu.VMEM((3, N1), jnp.float32),
            pltpu.VMEM((CPI, N1), jnp.float32),
        ],
        compiler_params=pltpu.CompilerParams(needs_layout_passes=False),
    )(_sc_gather_body)

    new_features = sc_call(h, idx3, w3, partial)
    return original_xyzs, new_features


# speedup vs baseline: 3.2555x; 1.1139x over previous
import functools

import jax
import jax.numpy as jnp
from jax import lax
from jax.experimental import pallas as pl
from jax.experimental.pallas import tpu as pltpu
from jax.experimental.pallas import tpu_sc as plsc

B = 8
L2 = 3
N2 = 512
L1 = 6
N1 = 2048
IN = 256
MID = 128
OUT = 128
ORIG = 64
BN_EPS = 1e-5

SC_NC = 2
SC_NS = 16
SC_NW = SC_NC * SC_NS
CPI = 16
N_SLOT = MID // CPI
N_ITEMS = B * L1 * N_SLOT
ITEMS_PW = N_ITEMS // SC_NW
N_GRP = N1 // 16


def _stage1_body(feat_ref, wt_ref, wmid_ref, gamma_ref, beta_ref, h_ref):
    wt = wt_ref[0]
    s1 = jnp.zeros((MID, 1), jnp.float32)
    s2 = jnp.zeros((MID, 1), jnp.float32)
    for b in range(B):
        g = jnp.dot(wt, feat_ref[b, 0], preferred_element_type=jnp.float32)
        h_ref[0, b] = g
        s1 = s1 + jnp.sum(g, axis=1, keepdims=True)
        s2 = s2 + jnp.sum(g * g, axis=1, keepdims=True)
    inv_n = jnp.float32(1.0 / (B * N2))
    mean = s1 * inv_n
    var = s2 * inv_n - mean * mean
    rstd = lax.rsqrt(var + BN_EPS)
    scale = gamma_ref[...] * rstd
    bias = beta_ref[...] - mean * scale
    wmid = wmid_ref[...]
    for b in range(B):
        sf = jnp.maximum(h_ref[0, b] * scale + bias, 0.0)
        h_ref[0, b] = jnp.dot(wmid, sf, preferred_element_type=jnp.float32)


def _stage2_body(seed_ref, anchor_ref, orig_ref, worig_ref,
                 idx_ref, w_ref, part_ref):
    s = seed_ref[0, 0]
    a = anchor_ref[0, 0]
    dx = s[:, 0:1] - a[0:1, :]
    dy = s[:, 1:2] - a[1:2, :]
    dz = s[:, 2:3] - a[2:3, :]
    d2 = (dx * dx + dy * dy) + dz * dz

    iota_i = lax.broadcasted_iota(jnp.int32, (1, N2), 1)
    mlow = (iota_i & 15).astype(jnp.float32)
    mhigh = (iota_i >> 4).astype(jnp.float32)
    recips = []
    for k in range(3):
        mn = jnp.min(d2, axis=0, keepdims=True)
        sel = d2 == mn
        recips.append(1.0 / (mn + 1e-8))
        sel_f = sel.astype(jnp.float32)
        il = jnp.dot(mlow, sel_f, preferred_element_type=jnp.float32)
        ih = jnp.dot(mhigh, sel_f, preferred_element_type=jnp.float32)
        idx_f = il + 16.0 * ih
        idx_i = (idx_f + 0.5).astype(jnp.int32)
        idx_ref[0, 0, k:k + 1, :] = jnp.minimum(idx_i, N2 - 1)
        d2 = jnp.where(sel, jnp.float32(jnp.inf), d2)

    norm = (recips[0] + recips[1]) + recips[2]
    for k in range(3):
        w_ref[0, 0, k:k + 1, :] = recips[k] / norm

    part_ref[0, 0] = jnp.dot(worig_ref[...], orig_ref[0, 0],
                             preferred_element_type=jnp.float32)


def _sc_gather_body(ht_hbm, idx_hbm, w_hbm, part_hbm, out_hbm,
                    table_a, idx_a, w_a, tile_a,
                    table_b, idx_b, w_b, tile_b, sem_a, sem_b):
    wid = lax.axis_index("s") * SC_NC + lax.axis_index("c")

    def coords(it):
        item = wid * ITEMS_PW + jnp.minimum(it, ITEMS_PW - 1)
        s = item // N_SLOT
        cslot = item % N_SLOT
        return s // L1, s % L1, cslot * CPI

    def copies(it, table_v, idx_v, w_v, tile_v, sem):
        b, j, c0 = coords(it)
        return (
            pltpu.make_async_copy(ht_hbm.at[j, b, pl.ds(c0, CPI), :],
                                  table_v, sem),
            pltpu.make_async_copy(idx_hbm.at[b, j], idx_v, sem),
            pltpu.make_async_copy(w_hbm.at[b, j], w_v, sem),
            pltpu.make_async_copy(part_hbm.at[b, j, pl.ds(c0, CPI), :],
                                  tile_v, sem),
        )

    def fetch(it, *bufs):
        for c in copies(it, *bufs):
            c.start()

    def drain(it, *bufs):
        for c in copies(it, *bufs):
            c.wait()

    def compute(it, table_v, idx_v, w_v, tile_v, sem):
        b, j, c0 = coords(it)

        @plsc.parallel_loop(0, N_GRP, 1, unroll=2)
        def grp_body(g):
            o = g * 16
            i0 = idx_v[0, pl.ds(o, 16)]
            i1 = idx_v[1, pl.ds(o, 16)]
            i2 = idx_v[2, pl.ds(o, 16)]
            w0 = w_v[0, pl.ds(o, 16)]
            w1 = w_v[1, pl.ds(o, 16)]
            w2 = w_v[2, pl.ds(o, 16)]
            for c in range(CPI):
                cc = jnp.full((16,), c, jnp.int32)
                v = w0 * plsc.load_gather(table_v, [cc, i0])
                v = v + w1 * plsc.load_gather(table_v, [cc, i1])
                v = v + w2 * plsc.load_gather(table_v, [cc, i2])
                plsc.addupdate(tile_v.at[c, pl.ds(o, 16)], v)

        pltpu.sync_copy(tile_v, out_hbm.at[b, j, pl.ds(c0, CPI), :])

    bufs_a = (table_a, idx_a, w_a, tile_a, sem_a)
    bufs_b = (table_b, idx_b, w_b, tile_b, sem_b)
    fetch(0, *bufs_a)
    fetch(1, *bufs_b)

    def pair_body(p, _):
        ia = 2 * p
        drain(ia, *bufs_a)
        compute(ia, *bufs_a)
        fetch(ia + 2, *bufs_a)
        drain(ia + 1, *bufs_b)
        compute(ia + 1, *bufs_b)
        fetch(ia + 3, *bufs_b)
        return 0

    lax.fori_loop(0, ITEMS_PW // 2, pair_body, 0)
    drain(ITEMS_PW, *bufs_a)
    drain(ITEMS_PW + 1, *bufs_b)


@jax.jit
def kernel(xyzs, original_xyzs, features, original_features, W_temporal,
           bn_gamma, bn_beta, W_spatial):
    w_taps = W_temporal.reshape(3, MID, IN)
    w_mid = W_spatial[:, :MID]
    w_orig = W_spatial[:, MID:]
    gamma = bn_gamma.reshape(MID, 1)
    beta = bn_beta.reshape(MID, 1)
    anchors_t = jnp.swapaxes(original_xyzs, 2, 3)

    h = pl.pallas_call(
        _stage1_body,
        grid=(L1,),
        in_specs=[
            pl.BlockSpec((B, 1, IN, N2), lambda j: (0, j // 2, 0, 0)),
            pl.BlockSpec((1, MID, IN), lambda j: (j % 2, 0, 0)),
            pl.BlockSpec((MID, MID), lambda j: (0, 0)),
            pl.BlockSpec((MID, 1), lambda j: (0, 0)),
            pl.BlockSpec((MID, 1), lambda j: (0, 0)),
        ],
        out_specs=pl.BlockSpec((1, B, MID, N2), lambda j: (j, 0, 0, 0)),
        out_shape=jax.ShapeDtypeStruct((L1, B, MID, N2), jnp.float32),
    )(features, w_taps, w_mid, gamma, beta)


    idx3, w3, partial = pl.pallas_call(
        _stage2_body,
        grid=(B, L1),
        in_specs=[
            pl.BlockSpec((1, 1, N2, 3), lambda b, j: (b, j // 2, 0, 0)),
            pl.BlockSpec((1, 1, 3, N1), lambda b, j: (b, j, 0, 0)),
            pl.BlockSpec((1, 1, ORIG, N1), lambda b, j: (b, j, 0, 0)),
            pl.BlockSpec((OUT, ORIG), lambda b, j: (0, 0)),
        ],
        out_specs=[
            pl.BlockSpec((1, 1, 3, N1), lambda b, j: (b, j, 0, 0)),
            pl.BlockSpec((1, 1, 3, N1), lambda b, j: (b, j, 0, 0)),
            pl.BlockSpec((1, 1, OUT, N1), lambda b, j: (b, j, 0, 0)),
        ],
        out_shape=[
            jax.ShapeDtypeStruct((B, L1, 3, N1), jnp.int32),
            jax.ShapeDtypeStruct((B, L1, 3, N1), jnp.float32),
            jax.ShapeDtypeStruct((B, L1, OUT, N1), jnp.float32),
        ],
    )(xyzs, anchors_t, original_features, w_orig)

    sc_call = functools.partial(
        pl.kernel,
        mesh=plsc.VectorSubcoreMesh(core_axis_name="c", subcore_axis_name="s"),
        out_type=jax.ShapeDtypeStruct((B, L1, OUT, N1), jnp.float32),
        scratch_types=[
            pltpu.VMEM((CPI, N2), jnp.float32),
            pltpu.VMEM((3, N1), jnp.int32),
            pltpu.VMEM((3, N1), jnp.float32),
            pltpu.VMEM((CPI, N1), jnp.float32),
            pltpu.VMEM((CPI, N2), jnp.float32),
            pltpu.VMEM((3, N1), jnp.int32),
            pltpu.VMEM((3, N1), jnp.float32),
            pltpu.VMEM((CPI, N1), jnp.float32),
            pltpu.SemaphoreType.DMA,
            pltpu.SemaphoreType.DMA,
        ],
        compiler_params=pltpu.CompilerParams(needs_layout_passes=False),
    )(_sc_gather_body)

    new_features = sc_call(h, idx3, w3, partial)
    return original_xyzs, new_features
